# baseline (speedup 1.0000x reference)
import functools

import jax
import jax.numpy as jnp
from jax import lax
from jax.experimental import pallas as pl
from jax.experimental.pallas import tpu as pltpu
from jax.experimental.pallas import tpu_sc as plsc

B, L, V, D = 1024, 200, 100000, 64
LANE = 16
NC, NS = 2, 16
NW = NC * NS
NCHK = B // (4 * LANE)
NBUF = 2
LH = L // 2
VCH = 4096
NCC = -(-V // VCH)
VP = VCH * NCC


def _pack_w(W):
    def body(lo_ref, hi_ref, out_ref):
        lo = lax.bitcast_convert_type(
            lo_ref[...].astype(jnp.bfloat16), jnp.uint16).astype(jnp.uint32)
        hi = lax.bitcast_convert_type(
            hi_ref[...].astype(jnp.bfloat16), jnp.uint16).astype(jnp.uint32)
        out_ref[...] = lax.bitcast_convert_type(
            lo | (hi << 16), jnp.int32).reshape(8 * VCH)

    w3 = W.reshape(8, 8, V)
    return pl.pallas_call(
        body,
        grid=(4, NCC),
        in_specs=[
            pl.BlockSpec((1, 8, VCH), lambda a, c: (a, 0, c)),
            pl.BlockSpec((1, 8, VCH), lambda a, c: (a + 4, 0, c)),
        ],
        out_specs=pl.BlockSpec((8 * VCH,), lambda a, c: (a * NCC + c,)),
        out_shape=jax.ShapeDtypeStruct((NW * VP,), jnp.int32),
    )(w3, w3)


def _pair_kernel(ids3, wpacked):
    mesh = plsc.VectorSubcoreMesh(core_axis_name="c", subcore_axis_name="s")

    @functools.partial(
        pl.kernel,
        out_type=jax.ShapeDtypeStruct((D, B), jnp.float32),
        mesh=mesh,
        compiler_params=pltpu.CompilerParams(
            needs_layout_passes=False, use_tc_tiling_on_sc=False),
        scratch_types=[
            pltpu.VMEM((VP,), jnp.int32),
            pltpu.VMEM((NBUF, LH * 64), jnp.int32),
            pltpu.VMEM((B,), jnp.float32),
            pltpu.VMEM((B,), jnp.float32),
            pltpu.SemaphoreType.DMA,
            pltpu.SemaphoreType.DMA,
            pltpu.SemaphoreType.DMA,
        ],
    )
    def k(ids_hbm, wp_hbm, out_hbm, wrow_v, chunk_v, out0_v, out1_v,
          sem0, sem1, wsem):
        sems = (sem0, sem1)
        wid = lax.axis_index("s") * NC + lax.axis_index("c")
        a, r = wid // 8, wid % 8
        base0 = (a * NCC * 8 + r) * VCH
        for i in range(NCC):
            pltpu.async_copy(
                wp_hbm.at[pl.ds(base0 + i * 8 * VCH, VCH)],
                wrow_v.at[pl.ds(i * VCH, VCH)], wsem)

        def issue(step, s):
            pltpu.async_copy(
                ids_hbm.at[pl.ds(step * LH * 64, LH * 64)],
                chunk_v.at[s], sems[s])

        nsteps = NCHK * 2
        issue(0, 0)
        pltpu.make_async_copy(
            wp_hbm.at[pl.ds(0, VP)], wrow_v, wsem).wait()
        for c in range(NCHK):
            accs = tuple(jnp.zeros((LANE,), jnp.float32) for _ in range(8))
            for h in range(2):
                step = c * 2 + h
                s = step % NBUF
                if step + 1 < nsteps:
                    issue(step + 1, (step + 1) % NBUF)
                pltpu.make_async_copy(
                    ids_hbm.at[pl.ds(0, LH * 64)], chunk_v.at[s],
                    sems[s]).wait()

                def jbody(j, accs):
                    new = []
                    for sg in range(4):
                        ids16 = chunk_v[s, pl.ds(j * 64 + sg * LANE, LANE)]
                        g = plsc.load_gather(wrow_v, [ids16])
                        v0, v1 = plsc.unpack(
                            plsc.bitcast(g, jnp.bfloat16),
                            format=plsc.PackFormat.INTERLEAVED)
                        new.append(accs[2 * sg] + v0)
                        new.append(accs[2 * sg + 1] + v1)
                    return tuple(new)

                accs = lax.fori_loop(0, LH, jbody, accs, unroll=2)
            for sg in range(4):
                out0_v[pl.ds(c * 64 + sg * LANE, LANE)] = accs[2 * sg]
                out1_v[pl.ds(c * 64 + sg * LANE, LANE)] = accs[2 * sg + 1]

        pltpu.sync_copy(out0_v, out_hbm.at[wid])
        pltpu.sync_copy(out1_v, out_hbm.at[wid + NW])

    return k(ids3, wpacked)


def kernel(token_ids, W, b):
    ids3 = (token_ids.astype(jnp.int32)
            .reshape(NCHK, 4 * LANE, L)
            .transpose(0, 2, 1)
            .reshape(-1))
    wpacked = _pack_w(W)
    out_t = _pair_kernel(ids3, wpacked)
    return (out_t.T + b[None, :])[:, None, :]


# speedup vs baseline: 1.1449x; 1.1449x over previous
import functools

import jax
import jax.numpy as jnp
from jax import lax
from jax.experimental import pallas as pl
from jax.experimental.pallas import tpu as pltpu
from jax.experimental.pallas import tpu_sc as plsc

B, L, V, D = 1024, 200, 100000, 64
LANE = 16
NC, NS = 2, 16
NW = NC * NS
NCHK = B // (4 * LANE)
NBUF = 2
LH = L // 2
VP = V


def _pair_kernel(ids3, wpacked):
    mesh = plsc.VectorSubcoreMesh(core_axis_name="c", subcore_axis_name="s")

    @functools.partial(
        pl.kernel,
        out_type=jax.ShapeDtypeStruct((D, B), jnp.float32),
        mesh=mesh,
        compiler_params=pltpu.CompilerParams(
            needs_layout_passes=False, use_tc_tiling_on_sc=False),
        scratch_types=[
            pltpu.VMEM((VP,), jnp.int32),
            pltpu.VMEM((NBUF, LH * 64), jnp.int32),
            pltpu.VMEM((B,), jnp.float32),
            pltpu.VMEM((B,), jnp.float32),
            pltpu.SemaphoreType.DMA,
            pltpu.SemaphoreType.DMA,
            pltpu.SemaphoreType.DMA,
        ],
    )
    def k(ids_hbm, wp_hbm, out_hbm, wrow_v, chunk_v, out0_v, out1_v,
          sem0, sem1, wsem):
        sems = (sem0, sem1)
        wid = lax.axis_index("s") * NC + lax.axis_index("c")
        pltpu.async_copy(wp_hbm.at[pl.ds(wid * VP, VP)], wrow_v, wsem)

        def issue(step, s):
            pltpu.async_copy(
                ids_hbm.at[pl.ds(step * LH * 64, LH * 64)],
                chunk_v.at[s], sems[s])

        nsteps = NCHK * 2
        issue(0, 0)
        pltpu.make_async_copy(
            wp_hbm.at[pl.ds(0, VP)], wrow_v, wsem).wait()
        for c in range(NCHK):
            accs = tuple(jnp.zeros((LANE,), jnp.float32) for _ in range(8))
            for h in range(2):
                step = c * 2 + h
                s = step % NBUF
                if step + 1 < nsteps:
                    issue(step + 1, (step + 1) % NBUF)
                pltpu.make_async_copy(
                    ids_hbm.at[pl.ds(0, LH * 64)], chunk_v.at[s],
                    sems[s]).wait()

                def jbody(j, accs):
                    new = []
                    for sg in range(4):
                        ids16 = chunk_v[s, pl.ds(j * 64 + sg * LANE, LANE)]
                        g = plsc.load_gather(wrow_v, [ids16])
                        v0, v1 = plsc.unpack(
                            plsc.bitcast(g, jnp.bfloat16),
                            format=plsc.PackFormat.INTERLEAVED)
                        new.append(accs[2 * sg] + v0)
                        new.append(accs[2 * sg + 1] + v1)
                    return tuple(new)

                accs = lax.fori_loop(0, LH, jbody, accs, unroll=5)
            for sg in range(4):
                out0_v[pl.ds(c * 64 + sg * LANE, LANE)] = accs[2 * sg]
                out1_v[pl.ds(c * 64 + sg * LANE, LANE)] = accs[2 * sg + 1]

        pltpu.sync_copy(out0_v, out_hbm.at[wid])
        pltpu.sync_copy(out1_v, out_hbm.at[wid + NW])

    return k(ids3, wpacked)


def kernel(token_ids, W, b):
    ids3 = (token_ids.astype(jnp.int32)
            .reshape(NCHK, 4 * LANE, L)
            .transpose(0, 2, 1)
            .reshape(-1))
    lo = lax.bitcast_convert_type(
        W[:NW].astype(jnp.bfloat16), jnp.uint16).astype(jnp.uint32)
    hi = lax.bitcast_convert_type(
        W[NW:].astype(jnp.bfloat16), jnp.uint16).astype(jnp.uint32)
    wpacked = lax.bitcast_convert_type(lo | (hi << 16), jnp.int32).reshape(-1)
    out_t = _pair_kernel(ids3, wpacked)
    return (out_t.T + b[None, :])[:, None, :]


# speedup vs baseline: 1.1689x; 1.0210x over previous
import functools

import jax
import jax.numpy as jnp
from jax import lax
from jax.experimental import pallas as pl
from jax.experimental.pallas import tpu as pltpu
from jax.experimental.pallas import tpu_sc as plsc

B, L, V, D = 1024, 200, 100000, 64
LANE = 16
NC, NS = 2, 16
NW = NC * NS
NCHK = B // (4 * LANE)
NBUF = 2
LH = L // 2
VP = V


def _pair_kernel(ids3, wpacked):
    mesh = plsc.VectorSubcoreMesh(core_axis_name="c", subcore_axis_name="s")

    @functools.partial(
        pl.kernel,
        out_type=jax.ShapeDtypeStruct((D, B), jnp.float32),
        mesh=mesh,
        compiler_params=pltpu.CompilerParams(
            needs_layout_passes=False, use_tc_tiling_on_sc=False),
        scratch_types=[
            pltpu.VMEM((VP,), jnp.int32),
            pltpu.VMEM((NBUF, LH * 64), jnp.int32),
            pltpu.VMEM((B,), jnp.float32),
            pltpu.VMEM((B,), jnp.float32),
            pltpu.SemaphoreType.DMA,
            pltpu.SemaphoreType.DMA,
            pltpu.SemaphoreType.DMA,
        ],
    )
    def k(ids_hbm, wp_hbm, out_hbm, wrow_v, chunk_v, out0_v, out1_v,
          sem0, sem1, wsem):
        sems = (sem0, sem1)
        wid = lax.axis_index("s") * NC + lax.axis_index("c")
        pltpu.async_copy(wp_hbm.at[pl.ds(wid * VP, VP)], wrow_v, wsem)

        def issue(step, s):
            pltpu.async_copy(
                ids_hbm.at[pl.ds(step * LH * 64, LH * 64)],
                chunk_v.at[s], sems[s])

        nsteps = NCHK * 2
        issue(0, 0)
        pltpu.make_async_copy(
            wp_hbm.at[pl.ds(0, VP)], wrow_v, wsem).wait()
        for c in range(NCHK):
            accs = tuple(jnp.zeros((LANE,), jnp.float32) for _ in range(8))
            for h in range(2):
                step = c * 2 + h
                s = step % NBUF
                if step + 1 < nsteps:
                    issue(step + 1, (step + 1) % NBUF)
                pltpu.make_async_copy(
                    ids_hbm.at[pl.ds(0, LH * 64)], chunk_v.at[s],
                    sems[s]).wait()

                def jbody(j, accs):
                    new = []
                    for sg in range(4):
                        ids16 = chunk_v[s, pl.ds(j * 64 + sg * LANE, LANE)]
                        g = plsc.load_gather(wrow_v, [ids16])
                        v0, v1 = plsc.unpack(
                            plsc.bitcast(g, jnp.bfloat16),
                            format=plsc.PackFormat.INTERLEAVED)
                        new.append(accs[2 * sg] + v0)
                        new.append(accs[2 * sg + 1] + v1)
                    return tuple(new)

                accs = lax.fori_loop(0, LH, jbody, accs, unroll=2)
            for sg in range(4):
                out0_v[pl.ds(c * 64 + sg * LANE, LANE)] = accs[2 * sg]
                out1_v[pl.ds(c * 64 + sg * LANE, LANE)] = accs[2 * sg + 1]

        pltpu.sync_copy(out0_v, out_hbm.at[wid])
        pltpu.sync_copy(out1_v, out_hbm.at[wid + NW])

    return k(ids3, wpacked)


def kernel(token_ids, W, b):
    ids3 = (token_ids.astype(jnp.int32)
            .reshape(NCHK, 4 * LANE, L)
            .transpose(0, 2, 1)
            .reshape(-1))
    lo = lax.bitcast_convert_type(
        W[:NW].astype(jnp.bfloat16), jnp.uint16).astype(jnp.uint32)
    hi = lax.bitcast_convert_type(
        W[NW:].astype(jnp.bfloat16), jnp.uint16).astype(jnp.uint32)
    wpacked = lax.bitcast_convert_type(lo | (hi << 16), jnp.int32).reshape(-1)
    out_t = _pair_kernel(ids3, wpacked)
    return (out_t.T + b[None, :])[:, None, :]


# speedup vs baseline: 1.2642x; 1.0815x over previous
import functools

import jax
import jax.numpy as jnp
from jax import lax
from jax.experimental import pallas as pl
from jax.experimental.pallas import tpu as pltpu
from jax.experimental.pallas import tpu_sc as plsc

B, L, V, D = 1024, 200, 100000, 64
LANE = 16
NC, NS = 2, 16
NW = NC * NS
NP = 16
RH = B // 2
NCHK = B // (4 * LANE)
CPW = NCHK // 2
NBUF = 2
LH = L // 2


def _half_kernel(ids3, wpacked_h):
    mesh = plsc.VectorSubcoreMesh(core_axis_name="c", subcore_axis_name="s")

    @functools.partial(
        pl.kernel,
        out_type=jax.ShapeDtypeStruct((2 * NP, B), jnp.float32),
        mesh=mesh,
        compiler_params=pltpu.CompilerParams(
            needs_layout_passes=False, use_tc_tiling_on_sc=False),
        scratch_types=[
            pltpu.VMEM((V,), jnp.int32),
            pltpu.VMEM((NBUF, LH * 64), jnp.int32),
            pltpu.VMEM((RH,), jnp.float32),
            pltpu.VMEM((RH,), jnp.float32),
            pltpu.SemaphoreType.DMA,
            pltpu.SemaphoreType.DMA,
            pltpu.SemaphoreType.DMA,
        ],
    )
    def k(ids_hbm, wp_hbm, out_hbm, wrow_v, chunk_v, out0_v, out1_v,
          sem0, sem1, wsem):
        sems = (sem0, sem1)
        wid = lax.axis_index("s") * NC + lax.axis_index("c")
        pair, rh = wid // 2, wid % 2
        pltpu.async_copy(wp_hbm.at[pl.ds(pair * V, V)], wrow_v, wsem)
        ids_base = rh * (CPW * 2 * LH * 64)

        def issue(step, s):
            pltpu.async_copy(
                ids_hbm.at[pl.ds(ids_base + step * LH * 64, LH * 64)],
                chunk_v.at[s], sems[s])

        nsteps = CPW * 2
        issue(0, 0)
        pltpu.make_async_copy(
            wp_hbm.at[pl.ds(0, V)], wrow_v, wsem).wait()
        for c in range(CPW):
            accs = tuple(jnp.zeros((LANE,), jnp.float32) for _ in range(8))
            for h in range(2):
                step = c * 2 + h
                s = step % NBUF
                if step + 1 < nsteps:
                    issue(step + 1, (step + 1) % NBUF)
                pltpu.make_async_copy(
                    ids_hbm.at[pl.ds(0, LH * 64)], chunk_v.at[s],
                    sems[s]).wait()

                def jbody(j, accs):
                    new = []
                    for sg in range(4):
                        ids16 = chunk_v[s, pl.ds(j * 64 + sg * LANE, LANE)]
                        g = plsc.load_gather(wrow_v, [ids16])
                        v0, v1 = plsc.unpack(
                            plsc.bitcast(g, jnp.bfloat16),
                            format=plsc.PackFormat.INTERLEAVED)
                        new.append(accs[2 * sg] + v0)
                        new.append(accs[2 * sg + 1] + v1)
                    return tuple(new)

                accs = lax.fori_loop(0, LH, jbody, accs, unroll=2)
            for sg in range(4):
                out0_v[pl.ds(c * 64 + sg * LANE, LANE)] = accs[2 * sg]
                out1_v[pl.ds(c * 64 + sg * LANE, LANE)] = accs[2 * sg + 1]

        pltpu.sync_copy(out0_v, out_hbm.at[pair, pl.ds(rh * RH, RH)])
        pltpu.sync_copy(out1_v, out_hbm.at[pair + NP, pl.ds(rh * RH, RH)])

    return k(ids3, wpacked_h)


def _pack_half(wlo, whi):
    lo = lax.bitcast_convert_type(
        wlo.astype(jnp.bfloat16), jnp.uint16).astype(jnp.uint32)
    hi = lax.bitcast_convert_type(
        whi.astype(jnp.bfloat16), jnp.uint16).astype(jnp.uint32)
    return lax.bitcast_convert_type(lo | (hi << 16), jnp.int32).reshape(-1)


def kernel(token_ids, W, b):
    ids3 = (token_ids.astype(jnp.int32)
            .reshape(NCHK, 4 * LANE, L)
            .transpose(0, 2, 1)
            .reshape(-1))
    wp_a = _pack_half(W[:NP], W[NW:NW + NP])
    wp_b = _pack_half(W[NP:NW], W[NW + NP:])
    out_a = _half_kernel(ids3, wp_a)
    out_b = _half_kernel(ids3, wp_b)
    out_t = jnp.concatenate(
        [out_a[:NP], out_b[:NP], out_a[NP:], out_b[NP:]], axis=0)
    return (out_t.T + b[None, :])[:, None, :]


# speedup vs baseline: 1.2681x; 1.0031x over previous
import functools

import jax
import jax.numpy as jnp
from jax import lax
from jax.experimental import pallas as pl
from jax.experimental.pallas import tpu as pltpu
from jax.experimental.pallas import tpu_sc as plsc

B, L, V, D = 1024, 200, 100000, 64
LANE = 16
NC, NS = 2, 16
NW = NC * NS
NP = 16
RH = B // 2
NCHK = B // (4 * LANE)
CPW = NCHK // 2
NBUF = 2
LH = L // 2


def _half_kernel(ids3, wpacked_h):
    mesh = plsc.VectorSubcoreMesh(core_axis_name="c", subcore_axis_name="s")

    @functools.partial(
        pl.kernel,
        out_type=jax.ShapeDtypeStruct((2 * NP, B), jnp.float32),
        mesh=mesh,
        compiler_params=pltpu.CompilerParams(
            needs_layout_passes=False, use_tc_tiling_on_sc=False),
        scratch_types=[
            pltpu.VMEM((V,), jnp.int32),
            pltpu.VMEM((NBUF, LH * 64), jnp.int32),
            pltpu.VMEM((RH,), jnp.float32),
            pltpu.VMEM((RH,), jnp.float32),
            pltpu.SemaphoreType.DMA,
            pltpu.SemaphoreType.DMA,
            pltpu.SemaphoreType.DMA,
        ],
    )
    def k(ids_hbm, wp_hbm, out_hbm, wrow_v, chunk_v, out0_v, out1_v,
          sem0, sem1, wsem):
        sems = (sem0, sem1)
        wid = lax.axis_index("s") * NC + lax.axis_index("c")
        pair, rh = wid // 2, wid % 2
        pltpu.async_copy(wp_hbm.at[pl.ds(pair * V, V)], wrow_v, wsem)
        ids_base = rh * (CPW * 2 * LH * 64)

        def issue(step, s):
            pltpu.async_copy(
                ids_hbm.at[pl.ds(ids_base + step * LH * 64, LH * 64)],
                chunk_v.at[s], sems[s])

        nsteps = CPW * 2
        issue(0, 0)
        pltpu.make_async_copy(
            wp_hbm.at[pl.ds(0, V)], wrow_v, wsem).wait()
        for c in range(CPW):
            accs = tuple(jnp.zeros((LANE,), jnp.float32) for _ in range(8))
            for h in range(2):
                step = c * 2 + h
                s = step % NBUF
                if step + 1 < nsteps:
                    issue(step + 1, (step + 1) % NBUF)
                pltpu.make_async_copy(
                    ids_hbm.at[pl.ds(0, LH * 64)], chunk_v.at[s],
                    sems[s]).wait()

                def jbody(j, accs):
                    new = []
                    for sg in range(4):
                        ids16 = chunk_v[s, pl.ds(j * 64 + sg * LANE, LANE)]
                        g = plsc.load_gather(wrow_v, [ids16])
                        v0, v1 = plsc.unpack(
                            plsc.bitcast(g, jnp.bfloat16),
                            format=plsc.PackFormat.INTERLEAVED)
                        new.append(accs[2 * sg] + v0)
                        new.append(accs[2 * sg + 1] + v1)
                    return tuple(new)

                accs = lax.fori_loop(0, LH, jbody, accs, unroll=2)
            for sg in range(4):
                out0_v[pl.ds(c * 64 + sg * LANE, LANE)] = accs[2 * sg]
                out1_v[pl.ds(c * 64 + sg * LANE, LANE)] = accs[2 * sg + 1]

        pltpu.sync_copy(out0_v, out_hbm.at[pair, pl.ds(rh * RH, RH)])
        pltpu.sync_copy(out1_v, out_hbm.at[pair + NP, pl.ds(rh * RH, RH)])

    return k(ids3, wpacked_h)


def _round_bf16_bits(w):
    u = lax.bitcast_convert_type(w, jnp.uint32)
    return (u + 0x7FFF + ((u >> 16) & 1)) >> 16


def _pack_half(wlo, whi):
    lo = _round_bf16_bits(wlo)
    hi = _round_bf16_bits(whi)
    return lax.bitcast_convert_type(lo | (hi << 16), jnp.int32).reshape(-1)


def kernel(token_ids, W, b):
    ids3 = (token_ids.astype(jnp.int32)
            .reshape(NCHK, 4 * LANE, L)
            .transpose(0, 2, 1)
            .reshape(-1))
    wp_a = _pack_half(W[:NP], W[NW:NW + NP])
    wp_b = _pack_half(W[NP:NW], W[NW + NP:])
    out_a = _half_kernel(ids3, wp_a)
    out_b = _half_kernel(ids3, wp_b)
    out_t = jnp.concatenate(
        [out_a[:NP], out_b[:NP], out_a[NP:], out_b[NP:]], axis=0)
    return (out_t.T + b[None, :])[:, None, :]
